# baseline (speedup 1.0000x reference)
import functools

import jax
import jax.numpy as jnp
from jax import lax
from jax.experimental import pallas as pl
from jax.experimental.pallas import tpu as pltpu
from jax.experimental.pallas import tpu_sc as plsc

_B, _C, _T = 16, 64, 1000
_K = 64
_KL = 9
_TV = _T - _KL + 1
_R = 8
_NB = 4
_DO = 64
_F = 2 * _K

_L = 128
_NCH = 8
_W = _L + _KL - 1
_TP = _NCH * _L + (_KL - 1)
_G = 32


def _feats_kernel(x_ref, kern_ref, thr_ref, mask_ref, tones_ref,
                  out_ref, m_ref):
    @pl.when(pl.program_id(0) == 0)
    def _build_m():
        d = (lax.broadcasted_iota(jnp.int32, (_W, _L), 0)
             - lax.broadcasted_iota(jnp.int32, (_W, _L), 1))
        diags = [(d == j).astype(jnp.bfloat16) for j in range(_KL)]
        for k in range(_K):
            tb = diags[0] * kern_ref[k, 0].astype(jnp.bfloat16)
            for j in range(1, _KL):
                tb = tb + diags[j] * kern_ref[k, j].astype(jnp.bfloat16)
            m_ref[:_W, k * _L:(k + 1) * _L] = tb
        m_ref[_W:, :] = -thr_ref[:, :]

    a = jnp.concatenate(
        [x_ref[:, ch * _L:ch * _L + _W] for ch in range(_NCH)], axis=0)
    ones_col = jnp.ones((_NCH * _G, 1), jnp.bfloat16)
    a = jnp.concatenate([a, ones_col], axis=1)
    conv = jnp.dot(a, m_ref[:, :],
                   preferred_element_type=jnp.float32).astype(jnp.bfloat16)
    nrows = (_NCH - 1) * _G
    conv_main = conv[:nrows, :]
    conv_last = conv[nrows:, :]
    big = jnp.bfloat16(3e38)
    one = jnp.bfloat16(1.0)
    zero = jnp.bfloat16(0.0)
    ind_main = jnp.minimum(jnp.maximum(conv_main, zero) * big, one)
    ind_last = (jnp.minimum(jnp.maximum(conv_last, zero) * big, one)
                * mask_ref[:, :])
    ind = jnp.concatenate([ind_main, ind_last], axis=0)
    counts = jnp.dot(ind, tones_ref[:, :], preferred_element_type=jnp.float32)
    total = counts[:_G, :]
    for ch in range(1, _NCH):
        total = total + counts[ch * _G:(ch + 1) * _G, :]
    ppv = total * (1.0 / _TV)
    mb = mask_ref[:, :] > zero
    neg = jnp.full(conv_last.shape, jnp.bfloat16(-3e38))
    conv_last_m = jnp.where(mb, conv_last, neg)
    m = conv_main[:_G, :]
    for ch in range(1, _NCH - 1):
        m = jnp.maximum(m, conv_main[ch * _G:(ch + 1) * _G, :])
    m = jnp.maximum(m, conv_last_m)
    m3 = m.reshape(_G, _K, _L)
    mx = jnp.max(m3, axis=-1).astype(jnp.float32)
    out_ref[:, :] = jnp.concatenate([ppv, mx], axis=1)


_CW = 128
_CH = _C // 2


def _sc_segsum_kernel(rid_hbm, feats_hbm, out_hbm, rid_v, buf_v, acc_v,
                      shared):
    cid = lax.axis_index("c")
    sid = lax.axis_index("s")
    col = sid * _CW
    pltpu.sync_copy(rid_hbm.at[pl.ds(cid * _CH, _CH)], rid_v)
    pltpu.sync_copy(feats_hbm.at[pl.ds(cid * _CH, _CH), pl.ds(col, _CW)],
                    buf_v)
    for r in range(_R):
        for v in range(_CW // 16):
            acc_v[r, v * 16:(v + 1) * 16] = jnp.zeros((16,), jnp.float32)
    pltpu.sync_copy(acc_v, shared.at[sid])
    pltpu.sync_copy(buf_v, shared.at[sid].at[rid_v], add=True)
    pltpu.sync_copy(shared.at[sid], out_hbm.at[cid, :, pl.ds(col, _CW)])


def _head_kernel(rid_ref, pooled_ref, w_ref, b_ref, out_ref):
    rid = rid_ref[:, :]
    rows = lax.broadcasted_iota(jnp.int32, (_R, _C), 0)
    m = (rid == rows).astype(jnp.float32)
    counts = jnp.maximum(jnp.sum(m, axis=1, keepdims=True), 1.0)
    pooled = (pooled_ref[0] + pooled_ref[1]) * (1.0 / counts)
    wc = jnp.concatenate([w_ref[n] for n in range(_NB)], axis=1)
    bc = jnp.concatenate([b_ref[n:n + 1, :] for n in range(_NB)], axis=1)
    for b in range(_B):
        pb = pooled[:, b * _F:(b + 1) * _F]
        ob = jnp.dot(pb, wc, preferred_element_type=jnp.float32)
        out_ref[b * _R:(b + 1) * _R, :] = ob + bc


def kernel(x, region_ids, kernels, biases, W, b):
    xr = x.reshape(_B * _C, _T)
    xp = jnp.pad(xr, ((0, 0), (0, _TP - _T))).astype(jnp.bfloat16)
    k2 = kernels.reshape(_K, _KL).astype(jnp.float32)

    thr = jnp.repeat(-biases.astype(jnp.float32), _L).reshape(1, _K * _L)
    thr = thr.astype(jnp.bfloat16)
    lane_t = jnp.arange(_K * _L) % _L
    maskv = (lane_t < (_TV - (_NCH - 1) * _L)).astype(jnp.bfloat16)
    maskv = maskv.reshape(1, _K * _L)
    tones = (jnp.arange(_K * _L)[:, None] // _L
             == jnp.arange(_K)[None, :]).astype(jnp.bfloat16)

    ncb = _C // _G
    feats_2d = pl.pallas_call(
        _feats_kernel,
        grid=(_B * _C // _G,),
        in_specs=[
            pl.BlockSpec((_G, _TP), lambda i: (i, 0)),
            pl.BlockSpec((_K, _KL), lambda i: (0, 0)),
            pl.BlockSpec((1, _K * _L), lambda i: (0, 0)),
            pl.BlockSpec((1, _K * _L), lambda i: (0, 0)),
            pl.BlockSpec((_K * _L, _K), lambda i: (0, 0)),
        ],
        out_specs=pl.BlockSpec((_G, _F), lambda i: (i % ncb, i // ncb)),
        out_shape=jax.ShapeDtypeStruct((_C, _B * _F), jnp.float32),
        scratch_shapes=[pltpu.VMEM((_W + 1, _K * _L), jnp.bfloat16)],
    )(xp, k2, thr, maskv, tones)

    rid = region_ids.astype(jnp.int32).reshape(1, _C)

    sc_segsum = functools.partial(
        pl.kernel,
        mesh=plsc.VectorSubcoreMesh(core_axis_name="c", subcore_axis_name="s"),
        out_type=jax.ShapeDtypeStruct((2, _R, _B * _F), jnp.float32),
        scratch_types=[
            pltpu.VMEM((_CH,), jnp.int32),
            pltpu.VMEM((_CH, _CW), jnp.float32),
            pltpu.VMEM((_R, _CW), jnp.float32),
            pltpu.VMEM_SHARED((16, _R, _CW), jnp.float32),
        ],
    )(_sc_segsum_kernel)
    pooled_sum = sc_segsum(region_ids.astype(jnp.int32), feats_2d)

    out = pl.pallas_call(
        _head_kernel,
        in_specs=[
            pl.BlockSpec((1, _C), lambda: (0, 0)),
            pl.BlockSpec((2, _R, _B * _F), lambda: (0, 0, 0)),
            pl.BlockSpec((_NB, _F, _DO), lambda: (0, 0, 0)),
            pl.BlockSpec((_NB, _DO), lambda: (0, 0)),
        ],
        out_specs=pl.BlockSpec((_B * _R, _NB * _DO), lambda: (0, 0)),
        out_shape=jax.ShapeDtypeStruct((_B * _R, _NB * _DO), jnp.float32),
    )(rid, pooled_sum, W, b)

    out = out.reshape(_B, _R, _NB, _DO).transpose(0, 2, 1, 3)
    return out.reshape(_B, _NB, _R * _DO)


# speedup vs baseline: 1.0134x; 1.0134x over previous
import functools

import jax
import jax.numpy as jnp
from jax import lax
from jax.experimental import pallas as pl
from jax.experimental.pallas import tpu as pltpu
from jax.experimental.pallas import tpu_sc as plsc

_B, _C, _T = 16, 64, 1000
_K = 64
_KL = 9
_TV = _T - _KL + 1
_R = 8
_NB = 4
_DO = 64
_F = 2 * _K

_L = 128
_NCH = 8
_W = _L + _KL - 1
_TP = _NCH * _L + (_KL - 1)
_G = 64


def _feats_kernel(x_ref, kern_ref, thr_ref, mask_ref, tones_ref,
                  out_ref, m_ref):
    @pl.when(pl.program_id(0) == 0)
    def _build_m():
        d = (lax.broadcasted_iota(jnp.int32, (_W, _L), 0)
             - lax.broadcasted_iota(jnp.int32, (_W, _L), 1))
        diags = [(d == j).astype(jnp.bfloat16) for j in range(_KL)]
        for k in range(_K):
            tb = diags[0] * kern_ref[k, 0].astype(jnp.bfloat16)
            for j in range(1, _KL):
                tb = tb + diags[j] * kern_ref[k, j].astype(jnp.bfloat16)
            m_ref[:_W, k * _L:(k + 1) * _L] = tb
        m_ref[_W:, :] = -thr_ref[:, :]

    a = jnp.concatenate(
        [x_ref[:, ch * _L:ch * _L + _W] for ch in range(_NCH)], axis=0)
    ones_col = jnp.ones((_NCH * _G, 1), jnp.bfloat16)
    a = jnp.concatenate([a, ones_col], axis=1)
    conv = jnp.dot(a, m_ref[:, :],
                   preferred_element_type=jnp.float32).astype(jnp.bfloat16)
    nrows = (_NCH - 1) * _G
    conv_main = conv[:nrows, :]
    conv_last = conv[nrows:, :]
    one = jnp.bfloat16(1.0)
    zero = jnp.bfloat16(0.0)
    ind_main = jnp.where(conv_main > zero, one, zero)
    ind_last = jnp.where(conv_last > zero, mask_ref[:, :], zero)
    ind = jnp.concatenate([ind_main, ind_last], axis=0)
    counts = jnp.dot(ind, tones_ref[:, :], preferred_element_type=jnp.float32)
    total = counts[:_G, :]
    for ch in range(1, _NCH):
        total = total + counts[ch * _G:(ch + 1) * _G, :]
    ppv = total * (1.0 / _TV)
    mb = mask_ref[:, :] > zero
    neg = jnp.full(conv_last.shape, jnp.bfloat16(-3e38))
    conv_last_m = jnp.where(mb, conv_last, neg)
    m = conv_main[:_G, :]
    for ch in range(1, _NCH - 1):
        m = jnp.maximum(m, conv_main[ch * _G:(ch + 1) * _G, :])
    m = jnp.maximum(m, conv_last_m)
    m3 = m.reshape(_G, _K, _L)
    mx = jnp.max(m3, axis=-1).astype(jnp.float32)
    out_ref[:, :] = jnp.concatenate([ppv, mx], axis=1)


_CW = 128
_CH = _C // 2


def _sc_segsum_kernel(rid_hbm, feats_hbm, out_hbm, rid_v, buf_v, acc_v,
                      shared):
    cid = lax.axis_index("c")
    sid = lax.axis_index("s")
    col = sid * _CW
    pltpu.sync_copy(rid_hbm.at[pl.ds(cid * _CH, _CH)], rid_v)
    pltpu.sync_copy(feats_hbm.at[pl.ds(cid * _CH, _CH), pl.ds(col, _CW)],
                    buf_v)
    for r in range(_R):
        for v in range(_CW // 16):
            acc_v[r, v * 16:(v + 1) * 16] = jnp.zeros((16,), jnp.float32)
    pltpu.sync_copy(acc_v, shared.at[sid])
    pltpu.sync_copy(buf_v, shared.at[sid].at[rid_v], add=True)
    pltpu.sync_copy(shared.at[sid], out_hbm.at[cid, :, pl.ds(col, _CW)])


def _head_kernel(rid_ref, pooled_ref, w_ref, b_ref, out_ref):
    rid = rid_ref[:, :]
    rows = lax.broadcasted_iota(jnp.int32, (_R, _C), 0)
    m = (rid == rows).astype(jnp.float32)
    counts = jnp.maximum(jnp.sum(m, axis=1, keepdims=True), 1.0)
    pooled = (pooled_ref[0] + pooled_ref[1]) * (1.0 / counts)
    wc = jnp.concatenate([w_ref[n] for n in range(_NB)], axis=1)
    bc = jnp.concatenate([b_ref[n:n + 1, :] for n in range(_NB)], axis=1)
    for b in range(_B):
        pb = pooled[:, b * _F:(b + 1) * _F]
        ob = jnp.dot(pb, wc, preferred_element_type=jnp.float32)
        out_ref[b * _R:(b + 1) * _R, :] = ob + bc


def kernel(x, region_ids, kernels, biases, W, b):
    xr = x.reshape(_B * _C, _T)
    xp = jnp.pad(xr, ((0, 0), (0, _TP - _T))).astype(jnp.bfloat16)
    k2 = kernels.reshape(_K, _KL).astype(jnp.float32)

    thr = jnp.repeat(-biases.astype(jnp.float32), _L).reshape(1, _K * _L)
    thr = thr.astype(jnp.bfloat16)
    lane_t = jnp.arange(_K * _L) % _L
    maskv = (lane_t < (_TV - (_NCH - 1) * _L)).astype(jnp.bfloat16)
    maskv = maskv.reshape(1, _K * _L)
    tones = (jnp.arange(_K * _L)[:, None] // _L
             == jnp.arange(_K)[None, :]).astype(jnp.bfloat16)

    ncb = _C // _G
    feats_2d = pl.pallas_call(
        _feats_kernel,
        grid=(_B * _C // _G,),
        in_specs=[
            pl.BlockSpec((_G, _TP), lambda i: (i, 0)),
            pl.BlockSpec((_K, _KL), lambda i: (0, 0)),
            pl.BlockSpec((1, _K * _L), lambda i: (0, 0)),
            pl.BlockSpec((1, _K * _L), lambda i: (0, 0)),
            pl.BlockSpec((_K * _L, _K), lambda i: (0, 0)),
        ],
        out_specs=pl.BlockSpec((_G, _F), lambda i: (i % ncb, i // ncb)),
        out_shape=jax.ShapeDtypeStruct((_C, _B * _F), jnp.float32),
        scratch_shapes=[pltpu.VMEM((_W + 1, _K * _L), jnp.bfloat16)],
    )(xp, k2, thr, maskv, tones)

    rid = region_ids.astype(jnp.int32).reshape(1, _C)

    sc_segsum = functools.partial(
        pl.kernel,
        mesh=plsc.VectorSubcoreMesh(core_axis_name="c", subcore_axis_name="s"),
        out_type=jax.ShapeDtypeStruct((2, _R, _B * _F), jnp.float32),
        scratch_types=[
            pltpu.VMEM((_CH,), jnp.int32),
            pltpu.VMEM((_CH, _CW), jnp.float32),
            pltpu.VMEM((_R, _CW), jnp.float32),
            pltpu.VMEM_SHARED((16, _R, _CW), jnp.float32),
        ],
    )(_sc_segsum_kernel)
    pooled_sum = sc_segsum(region_ids.astype(jnp.int32), feats_2d)

    out = pl.pallas_call(
        _head_kernel,
        in_specs=[
            pl.BlockSpec((1, _C), lambda: (0, 0)),
            pl.BlockSpec((2, _R, _B * _F), lambda: (0, 0, 0)),
            pl.BlockSpec((_NB, _F, _DO), lambda: (0, 0, 0)),
            pl.BlockSpec((_NB, _DO), lambda: (0, 0)),
        ],
        out_specs=pl.BlockSpec((_B * _R, _NB * _DO), lambda: (0, 0)),
        out_shape=jax.ShapeDtypeStruct((_B * _R, _NB * _DO), jnp.float32),
    )(rid, pooled_sum, W, b)

    out = out.reshape(_B, _R, _NB, _DO).transpose(0, 2, 1, 3)
    return out.reshape(_B, _NB, _R * _DO)
